# Initial kernel scaffold; baseline (speedup 1.0000x reference)
import jax
import jax.numpy as jnp
from jax.experimental import pallas as pl


def kernel(x, pos, W1a, b1a, W1b, b1b, W2a, b2a, W2b, b2b, edge_index):
    raise NotImplementedError("write your pallas kernel here")


# speedup vs baseline: 3.4453x; 3.4453x over previous
import functools

import jax
import jax.numpy as jnp
from jax import lax
from jax.experimental import pallas as pl
from jax.experimental.pallas import tpu as pltpu
from jax.experimental.pallas import tpu_sc as plsc

_N = 10000
_E = 320000
_F = 128
_H = 256

_NC = 2
_NS = 16
_NW = _NC * _NS

_EPW = _E // _NW
_CH = 80
_NCHUNK = _EPW // _CH

_ESC = _E // _NC
_EPT = _ESC // _NS
_CHS = 80
_NCH2 = _EPT // _CHS

_BE = 640


def _sc_gather(x, posp, row, col):
    mesh = plsc.VectorSubcoreMesh(core_axis_name="c", subcore_axis_name="s")

    @functools.partial(
        pl.kernel,
        out_type=[
            jax.ShapeDtypeStruct((_E, _F), jnp.float32),
            jax.ShapeDtypeStruct((_E, _F), jnp.float32),
            jax.ShapeDtypeStruct((_E, 8), jnp.float32),
            jax.ShapeDtypeStruct((_E, 8), jnp.float32),
        ],
        mesh=mesh,
        scratch_types=[
            pltpu.VMEM((_EPW,), jnp.int32),
            pltpu.VMEM((_EPW,), jnp.int32),
            pltpu.VMEM((_CH, _F), jnp.float32),
            pltpu.VMEM((_CH, _F), jnp.float32),
            pltpu.VMEM((_CH, 8), jnp.float32),
            pltpu.VMEM((_CH, 8), jnp.float32),
            pltpu.SemaphoreType.DMA,
            pltpu.SemaphoreType.DMA,
            pltpu.SemaphoreType.DMA,
            pltpu.SemaphoreType.DMA,
        ],
        compiler_params=pltpu.CompilerParams(use_tc_tiling_on_sc=False),
    )
    def gk(x_hbm, p_hbm, row_hbm, col_hbm, xr_hbm, xc_hbm, pr_hbm, pc_hbm,
           rbuf, cbuf, xrb, xcb, prb, pcb, s0, s1, s2, s3):
        wid = lax.axis_index("s") * _NC + lax.axis_index("c")
        base = wid * _EPW
        pltpu.sync_copy(row_hbm.at[pl.ds(base, _EPW)], rbuf)
        pltpu.sync_copy(col_hbm.at[pl.ds(base, _EPW)], cbuf)

        def body(i, carry):
            off = i * _CH
            ir = rbuf.at[pl.ds(off, _CH)]
            ic = cbuf.at[pl.ds(off, _CH)]
            c0 = pltpu.async_copy(x_hbm.at[ir], xrb, s0)
            c1 = pltpu.async_copy(x_hbm.at[ic], xcb, s1)
            c2 = pltpu.async_copy(p_hbm.at[ir], prb, s2)
            c3 = pltpu.async_copy(p_hbm.at[ic], pcb, s3)
            c0.wait()
            c1.wait()
            c2.wait()
            c3.wait()
            pltpu.sync_copy(xrb, xr_hbm.at[pl.ds(base + off, _CH)])
            pltpu.sync_copy(xcb, xc_hbm.at[pl.ds(base + off, _CH)])
            pltpu.sync_copy(prb, pr_hbm.at[pl.ds(base + off, _CH)])
            pltpu.sync_copy(pcb, pc_hbm.at[pl.ds(base + off, _CH)])
            return carry

        lax.fori_loop(0, _NCHUNK, body, 0)

    return gk(x, posp, row, col)


def _sc_scatter(vec, col2d, zeros):
    mesh = plsc.VectorSubcoreMesh(core_axis_name="c", subcore_axis_name="s")

    @functools.partial(
        pl.kernel,
        out_type=jax.ShapeDtypeStruct((_NC, _N, 8), jnp.float32),
        mesh=mesh,
        scratch_types=[
            pltpu.VMEM((_NCH2, _CHS), jnp.int32),
            pltpu.VMEM((_CHS, 8), jnp.float32),
            pltpu.VMEM_SHARED((_N, 8), jnp.float32),
        ],
        compiler_params=pltpu.CompilerParams(use_tc_tiling_on_sc=False),
    )
    def sk(vec_hbm, col_hbm, z_hbm, out_hbm, idxb, vbuf, acc):
        c = lax.axis_index("c")
        s = lax.axis_index("s")

        @pl.when(s == 0)
        def _():
            pltpu.sync_copy(z_hbm, acc)

        plsc.subcore_barrier()
        rowstart = c * (_ESC // _CHS) + s * _NCH2
        pltpu.sync_copy(col_hbm.at[pl.ds(rowstart, _NCH2)], idxb)
        base = c * _ESC + s * _EPT

        def body(j, carry):
            pltpu.sync_copy(vec_hbm.at[pl.ds(base + j * _CHS, _CHS)], vbuf)
            pltpu.sync_copy(vbuf, acc.at[idxb.at[j]], add=True)
            return carry

        lax.fori_loop(0, _NCH2, body, 0)
        plsc.subcore_barrier()

        @pl.when(s == 0)
        def _():
            pltpu.sync_copy(acc, out_hbm.at[c])

    return sk(vec, col2d, zeros)


def _edge_body(xr_ref, xc_ref, pr_ref, pc_ref, wsrc_ref, wdst_ref, wd_ref,
               b_ref, wb_ref, bb_ref, out_ref):
    d8 = pr_ref[...] - pc_ref[...]
    d2 = d8 * d8
    dist = jnp.sqrt(d2[:, 0:1] + d2[:, 1:2] + d2[:, 2:3])
    h = jnp.dot(xr_ref[...], wsrc_ref[...], preferred_element_type=jnp.float32)
    h = h + jnp.dot(xc_ref[...], wdst_ref[...], preferred_element_type=jnp.float32)
    h = h + dist * wd_ref[...] + b_ref[...]
    sh = h * jax.nn.sigmoid(h)
    mes = jnp.dot(sh, wb_ref[...], preferred_element_type=jnp.float32) + bb_ref[...]
    r = dist * (1.0 / 4.5)
    r2 = r * r
    r5 = r2 * r2 * r
    coe = 1.0 - 21.0 * r5 + 35.0 * r5 * r - 15.0 * r5 * r2
    sc = coe / (dist + 1e-8)
    s1 = sc * mes[:, 0:1]
    s2 = sc * mes[:, 1:2]
    dv = d8[:, 0:4]
    out_ref[...] = jnp.concatenate([dv * s1, dv * s2], axis=1)


def _tc_edge(xr, xc, pr, pc, wsrc, wdst, wd, bias, wb, bb):
    grid = (_E // _BE,)
    full = lambda i: (0, 0)
    blk = lambda i: (i, 0)
    return pl.pallas_call(
        _edge_body,
        grid=grid,
        in_specs=[
            pl.BlockSpec((_BE, _F), blk),
            pl.BlockSpec((_BE, _F), blk),
            pl.BlockSpec((_BE, 8), blk),
            pl.BlockSpec((_BE, 8), blk),
            pl.BlockSpec((_F, 2 * _H), full),
            pl.BlockSpec((_F, 2 * _H), full),
            pl.BlockSpec((1, 2 * _H), full),
            pl.BlockSpec((1, 2 * _H), full),
            pl.BlockSpec((2 * _H, 8), full),
            pl.BlockSpec((1, 8), full),
        ],
        out_specs=pl.BlockSpec((_BE, 8), blk),
        out_shape=jax.ShapeDtypeStruct((_E, 8), jnp.float32),
    )(xr, xc, pr, pc, wsrc, wdst, wd, bias, wb, bb)


def _final_body(p0_ref, p1_ref, out_ref):
    v = p0_ref[...] + p1_ref[...]
    v1 = v[:, 0:4]
    v2 = v[:, 4:8]
    q1 = v1 * v1
    n1 = v1 / (jnp.sqrt(q1[:, 0:1] + q1[:, 1:2] + q1[:, 2:3]) + 1e-8)
    t = n1 * v2
    sdot = t[:, 0:1] + t[:, 1:2] + t[:, 2:3]
    n2p = v2 - sdot * n1
    q2 = n2p * n2p
    n2 = n2p / (jnp.sqrt(q2[:, 0:1] + q2[:, 1:2] + q2[:, 2:3]) + 1e-8)
    n3x = n1[:, 1:2] * n2[:, 2:3] - n1[:, 2:3] * n2[:, 1:2]
    n3y = n1[:, 2:3] * n2[:, 0:1] - n1[:, 0:1] * n2[:, 2:3]
    n3z = n1[:, 0:1] * n2[:, 1:2] - n1[:, 1:2] * n2[:, 0:1]
    out_ref[...] = jnp.concatenate(
        [n1[:, 0:3], n2[:, 0:3], n3x, n3y, n3z], axis=1)


def _tc_final(p0, p1):
    return pl.pallas_call(
        _final_body,
        out_shape=jax.ShapeDtypeStruct((_N, 9), jnp.float32),
    )(p0, p1)


def kernel(x, pos, W1a, b1a, W1b, b1b, W2a, b2a, W2b, b2b, edge_index):
    x = x.astype(jnp.float32)
    row = edge_index[0].astype(jnp.int32)
    col = edge_index[1].astype(jnp.int32)
    posp = jnp.pad(pos.astype(jnp.float32), ((0, 0), (0, 5)))

    wsrc = jnp.concatenate([W1a[:_F], W2a[:_F]], axis=1)
    wdst = jnp.concatenate([W1a[_F:2 * _F], W2a[_F:2 * _F]], axis=1)
    wd = jnp.concatenate([W1a[2 * _F], W2a[2 * _F]])[None, :]
    bias = jnp.concatenate([b1a, b2a])[None, :]
    wb = jnp.zeros((2 * _H, 8), jnp.float32)
    wb = wb.at[:_H, 0].set(W1b[:, 0]).at[_H:, 1].set(W2b[:, 0])
    bb = jnp.zeros((1, 8), jnp.float32).at[0, 0].set(b1b[0]).at[0, 1].set(b2b[0])

    xr, xc, pr, pc = _sc_gather(x, posp, row, col)
    vec = _tc_edge(xr, xc, pr, pc, wsrc, wdst, wd, bias, wb, bb)
    part = _sc_scatter(vec, col.reshape(_E // _CHS, _CHS),
                       jnp.zeros((_N, 8), jnp.float32))
    out9 = _tc_final(part[0], part[1])
    return out9.reshape(_N, 3, 3)
